# baseline (speedup 1.0000x reference)
import functools

import jax
import jax.numpy as jnp
from jax import lax
from jax.experimental import pallas as pl
from jax.experimental.pallas import tpu as pltpu
from jax.experimental.pallas import tpu_sc as plsc

B = 16384
D = 128
NV = 100
NC = 2
NS = 16
NW = NC * NS
B_PER_W = B // NW
R = 128
N_CHUNKS = B_PER_W // R
N_XBUF = 4

_mesh = plsc.VectorSubcoreMesh(core_axis_name="c", subcore_axis_name="s")

_scratch = (
    [pltpu.VMEM((B_PER_W,), jnp.int32),
     pltpu.VMEM_SHARED((NV, D), jnp.float32)]
    + [pltpu.VMEM((R, D), jnp.float32) for _ in range(N_XBUF)]
    + [pltpu.SemaphoreType.DMA for _ in range(N_XBUF)]
    + [pltpu.SemaphoreType.DMA for _ in range(N_XBUF)]
    + [pltpu.SemaphoreType.DMA for _ in range(N_XBUF)]
)


@functools.partial(
    pl.kernel,
    mesh=_mesh,
    out_type=jax.ShapeDtypeStruct((B, D), jnp.float32),
    scratch_types=_scratch,
)
def _sc_embed_add(x_hbm, idx_hbm, emb_hbm, out_hbm, idx_all, emb_sh, *bufs):
    x_v = bufs[:N_XBUF]
    semx = bufs[N_XBUF:2 * N_XBUF]
    semg = bufs[2 * N_XBUF:3 * N_XBUF]
    semo = bufs[3 * N_XBUF:]

    wid = lax.axis_index("s") * NC + lax.axis_index("c")
    base = wid * B_PER_W
    sid = lax.axis_index("s")

    @pl.when(sid == 0)
    def _():
        pltpu.sync_copy(emb_hbm, emb_sh)

    pltpu.sync_copy(idx_hbm.at[pl.ds(base, B_PER_W)], idx_all)
    plsc.subcore_barrier()

    x_descs = [None for _ in range(N_CHUNKS)]
    out_descs = [None for _ in range(N_XBUF)]

    def issue_x(ch):
        b = ch % N_XBUF
        if out_descs[b] is not None:
            out_descs[b].wait()
        x_descs[ch] = pltpu.async_copy(x_hbm.at[pl.ds(base + ch * R, R)],
                                       x_v[b], semx[b])

    issue_x(0)
    for ch in range(N_CHUNKS):
        b = ch % N_XBUF
        if ch + 1 < N_CHUNKS:
            issue_x(ch + 1)
        x_descs[ch].wait()
        g = pltpu.async_copy(emb_sh.at[idx_all.at[pl.ds(ch * R, R)]],
                             x_v[b], semg[b], add=True)
        g.wait()
        out_descs[b] = pltpu.async_copy(x_v[b],
                                        out_hbm.at[pl.ds(base + ch * R, R)],
                                        semo[b])
    for d in out_descs:
        if d is not None:
            d.wait()


def kernel(x, condition_idx, embeddings):
    idx = condition_idx.astype(jnp.int32)
    return _sc_embed_add(x, idx, embeddings)


# speedup vs baseline: 1.0089x; 1.0089x over previous
import functools

import jax
import jax.numpy as jnp
from jax import lax
from jax.experimental import pallas as pl
from jax.experimental.pallas import tpu as pltpu
from jax.experimental.pallas import tpu_sc as plsc

B = 16384
D = 128
NV = 100
NC = 2
NS = 16
NW = NC * NS
B_PER_W = B // NW
R = 128
N_CHUNKS = B_PER_W // R
N_XBUF = 3

_mesh = plsc.VectorSubcoreMesh(core_axis_name="c", subcore_axis_name="s")

_scratch = (
    [pltpu.VMEM((B_PER_W,), jnp.int32),
     pltpu.VMEM_SHARED((NV, D), jnp.float32)]
    + [pltpu.VMEM((R, D), jnp.float32) for _ in range(N_XBUF)]
    + [pltpu.SemaphoreType.DMA for _ in range(N_XBUF)]
    + [pltpu.SemaphoreType.DMA for _ in range(N_XBUF)]
    + [pltpu.SemaphoreType.DMA for _ in range(N_XBUF)]
)


@functools.partial(
    pl.kernel,
    mesh=_mesh,
    out_type=jax.ShapeDtypeStruct((B, D), jnp.float32),
    scratch_types=_scratch,
)
def _sc_embed_add(x_hbm, idx_hbm, emb_hbm, out_hbm, idx_all, emb_sh, *bufs):
    x_v = bufs[:N_XBUF]
    semx = bufs[N_XBUF:2 * N_XBUF]
    semg = bufs[2 * N_XBUF:3 * N_XBUF]
    semo = bufs[3 * N_XBUF:]

    wid = lax.axis_index("s") * NC + lax.axis_index("c")
    base = wid * B_PER_W
    sid = lax.axis_index("s")

    @pl.when(sid == 0)
    def _():
        pltpu.sync_copy(emb_hbm, emb_sh)

    pltpu.sync_copy(idx_hbm.at[pl.ds(base, B_PER_W)], idx_all)
    plsc.subcore_barrier()

    x_descs = [None for _ in range(N_CHUNKS)]
    out_descs = [None for _ in range(N_XBUF)]

    def issue_x(ch):
        b = ch % N_XBUF
        if out_descs[b] is not None:
            out_descs[b].wait()
        x_descs[ch] = pltpu.async_copy(x_hbm.at[pl.ds(base + ch * R, R)],
                                       x_v[b], semx[b])

    issue_x(0)
    for ch in range(N_CHUNKS):
        b = ch % N_XBUF
        if ch + 1 < N_CHUNKS:
            issue_x(ch + 1)
        x_descs[ch].wait()
        g = pltpu.async_copy(emb_sh.at[idx_all.at[pl.ds(ch * R, R)]],
                             x_v[b], semg[b], add=True)
        g.wait()
        out_descs[b] = pltpu.async_copy(x_v[b],
                                        out_hbm.at[pl.ds(base + ch * R, R)],
                                        semo[b])
    for d in out_descs:
        if d is not None:
            d.wait()


def kernel(x, condition_idx, embeddings):
    idx = condition_idx.astype(jnp.int32)
    return _sc_embed_add(x, idx, embeddings)
